# Initial kernel scaffold; baseline (speedup 1.0000x reference)
import jax
import jax.numpy as jnp
from jax.experimental import pallas as pl


def kernel(x, edge_attr, node_table, edge_table, node_gamma, node_beta, edge_gamma, edge_beta):
    raise NotImplementedError("write your pallas kernel here")


# speedup vs baseline: 3.1161x; 3.1161x over previous
import functools

import jax
import jax.numpy as jnp
from jax import lax
from jax.experimental import pallas as pl
from jax.experimental.pallas import tpu as pltpu
from jax.experimental.pallas import tpu_sc as plsc

N_NODES = 50000
N_EDGES = 800000
D = 64
EDGE_VOCAB = 1000
EV_PAD = 1024
EPS = 1e-5

NC, NS = 2, 16
NW = NC * NS

NP = 50176
N_PER_W = NP // NW
CH_N = 392
E_PER_W = N_EDGES // NW
CH_E = 1000
HIST_W = 16

_mesh = plsc.VectorSubcoreMesh(
    core_axis_name="c", subcore_axis_name="s", num_cores=NC, num_subcores=NS
)
_sc_params = pltpu.CompilerParams(use_tc_tiling_on_sc=False)


def _hist_body(ea_hbm, counts_hbm, idx_v, ones_v, stripe_v, shared, sem):
    cid = lax.axis_index("c")
    sid = lax.axis_index("s")
    rows = EV_PAD // NS

    @pl.loop(0, rows)
    def _(i):
        stripe_v[i, :] = jnp.zeros((HIST_W,), jnp.float32)

    pltpu.sync_copy(stripe_v, shared.at[pl.ds(sid * rows, rows)])

    @pl.loop(0, CH_E)
    def _(i):
        ones_v[i, :] = jnp.full((HIST_W,), 1.0, jnp.float32)

    plsc.subcore_barrier()

    base0 = cid * (N_EDGES // NC) + sid * E_PER_W

    @pl.loop(0, E_PER_W // CH_E)
    def _(k):
        pltpu.sync_copy(ea_hbm.at[pl.ds(base0 + k * CH_E, CH_E)], idx_v)
        pltpu.sync_copy(ones_v, shared.at[idx_v], add=True)

    plsc.subcore_barrier()

    pltpu.sync_copy(
        shared.at[pl.ds(sid * rows, rows)],
        counts_hbm.at[pl.ds(cid * EV_PAD + sid * rows, rows)],
    )


@functools.partial(
    pl.kernel,
    out_type=jax.ShapeDtypeStruct((NC * EV_PAD, HIST_W), jnp.float32),
    mesh=_mesh,
    scratch_types=[
        pltpu.VMEM((CH_E,), jnp.int32),
        pltpu.VMEM((CH_E, HIST_W), jnp.float32),
        pltpu.VMEM((EV_PAD // NS, HIST_W), jnp.float32),
        pltpu.VMEM_SHARED((EV_PAD, HIST_W), jnp.float32),
        pltpu.SemaphoreType.DMA,
    ],
    compiler_params=_sc_params,
)
def _hist_kernel(ea_hbm, counts_hbm, idx_v, ones_v, stripe_v, shared, sem):
    _hist_body(ea_hbm, counts_hbm, idx_v, ones_v, stripe_v, shared, sem)


def _edge_table_body(counts_ref, t_ref, g_ref, b_ref, out_ref):
    c = counts_ref[0:EV_PAD, :] + counts_ref[EV_PAD : 2 * EV_PAD, :]
    cc = c[:EDGE_VOCAB, 0:1]
    t = t_ref[...]
    s1 = jnp.sum(cc * t, axis=0, keepdims=True)
    s2 = jnp.sum(cc * t * t, axis=0, keepdims=True)
    inv_n = jnp.float32(1.0 / N_EDGES)
    mean = s1 * inv_n
    var = s2 * inv_n - mean * mean
    a = g_ref[...] * lax.rsqrt(var + EPS)
    b = b_ref[...] - mean * a
    out_ref[...] = t * a + b


def _edge_table(counts, table, gamma, beta):
    return pl.pallas_call(
        _edge_table_body,
        out_shape=jax.ShapeDtypeStruct((EDGE_VOCAB, D), jnp.float32),
    )(counts, table, gamma.reshape(1, D), beta.reshape(1, D))


def _gather_body(
    etab_hbm, ea_hbm, ntab_hbm, xp_hbm, he_hbm, hn_hbm,
    idxe_v, rowe_v, idxn_v, rown_v, sem
):
    cid = lax.axis_index("c")
    sid = lax.axis_index("s")
    wid = sid * NC + cid

    nbase = wid * N_PER_W

    @pl.loop(0, N_PER_W // CH_N)
    def _(k):
        b = nbase + k * CH_N
        pltpu.sync_copy(xp_hbm.at[pl.ds(b, CH_N)], idxn_v)
        pltpu.async_copy(ntab_hbm.at[idxn_v], rown_v, sem).wait()
        pltpu.sync_copy(rown_v, hn_hbm.at[pl.ds(b, CH_N)])

    ebase = wid * E_PER_W

    @pl.loop(0, E_PER_W // CH_E)
    def _(k):
        b = ebase + k * CH_E
        pltpu.sync_copy(ea_hbm.at[pl.ds(b, CH_E)], idxe_v)
        pltpu.async_copy(etab_hbm.at[idxe_v], rowe_v, sem).wait()
        pltpu.sync_copy(rowe_v, he_hbm.at[pl.ds(b, CH_E)])


@functools.partial(
    pl.kernel,
    out_type=(
        jax.ShapeDtypeStruct((N_EDGES, D), jnp.float32),
        jax.ShapeDtypeStruct((NP, D), jnp.float32),
    ),
    mesh=_mesh,
    scratch_types=[
        pltpu.VMEM((CH_E,), jnp.int32),
        pltpu.VMEM((CH_E, D), jnp.float32),
        pltpu.VMEM((CH_N,), jnp.int32),
        pltpu.VMEM((CH_N, D), jnp.float32),
        pltpu.SemaphoreType.DMA,
    ],
    compiler_params=_sc_params,
)
def _gather_kernel(*refs):
    _gather_body(*refs)


def _node_bn_body(h_ref, g_ref, b_ref, out_ref):
    h = h_ref[...]
    mask = (
        lax.broadcasted_iota(jnp.int32, (NP, 1), 0) < N_NODES
    ).astype(jnp.float32)
    hm = h * mask
    s1 = jnp.sum(hm, axis=0, keepdims=True)
    s2 = jnp.sum(hm * hm, axis=0, keepdims=True)
    inv_n = jnp.float32(1.0 / N_NODES)
    mean = s1 * inv_n
    var = s2 * inv_n - mean * mean
    a = g_ref[...] * lax.rsqrt(var + EPS)
    b = b_ref[...] - mean * a
    out_ref[...] = h[:N_NODES, :] * a + b


def _node_bn(h_raw, gamma, beta):
    return pl.pallas_call(
        _node_bn_body,
        out_shape=jax.ShapeDtypeStruct((N_NODES, D), jnp.float32),
    )(h_raw, gamma.reshape(1, D), beta.reshape(1, D))


def kernel(x, edge_attr, node_table, edge_table,
           node_gamma, node_beta, edge_gamma, edge_beta):
    x = x.astype(jnp.int32)
    edge_attr = edge_attr.astype(jnp.int32)
    x_pad = jnp.pad(x, (0, NP - N_NODES))

    counts = _hist_kernel(edge_attr)
    norm_etab = _edge_table(counts, edge_table, edge_gamma, edge_beta)
    h_edge, h_node_raw = _gather_kernel(norm_etab, edge_attr, node_table, x_pad)
    h_node = _node_bn(h_node_raw, node_gamma, node_beta)
    return (h_node, h_edge)


# speedup vs baseline: 3.7254x; 1.1955x over previous
import functools

import jax
import jax.numpy as jnp
from jax import lax
from jax.experimental import pallas as pl
from jax.experimental.pallas import tpu as pltpu
from jax.experimental.pallas import tpu_sc as plsc
from jax.experimental import layout as jlayout

N_NODES = 50000
N_EDGES = 800000
D = 64
EDGE_VOCAB = 1000
EV_PAD = 1024
EPS = 1e-5

NC, NS = 2, 16
NW = NC * NS

NP = 50176
N_PER_W = NP // NW
CH_N = 392
E_PER_W = N_EDGES // NW
CH_E = 1000
HIST_W = 16

_mesh = plsc.VectorSubcoreMesh(
    core_axis_name="c", subcore_axis_name="s", num_cores=NC, num_subcores=NS
)
_sc_params = pltpu.CompilerParams(use_tc_tiling_on_sc=False)


def _hist_body(ea_hbm, counts_hbm, idx_v, ones_v, stripe_v, shared, sem):
    cid = lax.axis_index("c")
    sid = lax.axis_index("s")
    rows = EV_PAD // NS

    @pl.loop(0, rows)
    def _(i):
        stripe_v[i, :] = jnp.zeros((HIST_W,), jnp.float32)

    pltpu.sync_copy(stripe_v, shared.at[pl.ds(sid * rows, rows)])

    @pl.loop(0, CH_E)
    def _(i):
        ones_v[i, :] = jnp.full((HIST_W,), 1.0, jnp.float32)

    plsc.subcore_barrier()

    base0 = cid * (N_EDGES // NC) + sid * E_PER_W

    @pl.loop(0, E_PER_W // CH_E)
    def _(k):
        pltpu.sync_copy(ea_hbm.at[pl.ds(base0 + k * CH_E, CH_E)], idx_v)
        pltpu.sync_copy(ones_v, shared.at[idx_v], add=True)

    plsc.subcore_barrier()

    pltpu.sync_copy(
        shared.at[pl.ds(sid * rows, rows)],
        counts_hbm.at[pl.ds(cid * EV_PAD + sid * rows, rows)],
    )


@functools.partial(
    pl.kernel,
    out_type=jax.ShapeDtypeStruct((NC * EV_PAD, HIST_W), jnp.float32),
    mesh=_mesh,
    scratch_types=[
        pltpu.VMEM((CH_E,), jnp.int32),
        pltpu.VMEM((CH_E, HIST_W), jnp.float32),
        pltpu.VMEM((EV_PAD // NS, HIST_W), jnp.float32),
        pltpu.VMEM_SHARED((EV_PAD, HIST_W), jnp.float32),
        pltpu.SemaphoreType.DMA,
    ],
    compiler_params=_sc_params,
)
def _hist_kernel(ea_hbm, counts_hbm, idx_v, ones_v, stripe_v, shared, sem):
    _hist_body(ea_hbm, counts_hbm, idx_v, ones_v, stripe_v, shared, sem)


def _edge_table_body(counts_ref, t_ref, g_ref, b_ref, out_ref):
    c = counts_ref[0:EV_PAD, :] + counts_ref[EV_PAD : 2 * EV_PAD, :]
    cc = c[:EDGE_VOCAB, 0:1]
    t = t_ref[...]
    s1 = jnp.sum(cc * t, axis=0, keepdims=True)
    s2 = jnp.sum(cc * t * t, axis=0, keepdims=True)
    inv_n = jnp.float32(1.0 / N_EDGES)
    mean = s1 * inv_n
    var = s2 * inv_n - mean * mean
    a = g_ref[...] * lax.rsqrt(var + EPS)
    b = b_ref[...] - mean * a
    out_ref[...] = t * a + b


def _edge_table(counts, table, gamma, beta):
    return pl.pallas_call(
        _edge_table_body,
        out_shape=jax.ShapeDtypeStruct((EDGE_VOCAB, D), jnp.float32),
    )(counts, table, gamma.reshape(1, D), beta.reshape(1, D))


def _gather_body(
    etab_hbm, ea_hbm, ntab_hbm, xp_hbm, he_hbm, hn_hbm,
    idxe_v, rowe_v, idxn_v, rown_v, sem
):
    cid = lax.axis_index("c")
    sid = lax.axis_index("s")
    wid = sid * NC + cid

    nbase = wid * N_PER_W

    @pl.loop(0, N_PER_W // CH_N)
    def _(k):
        b = nbase + k * CH_N
        pltpu.sync_copy(xp_hbm.at[pl.ds(b, CH_N)], idxn_v)
        pltpu.async_copy(ntab_hbm.at[idxn_v], rown_v, sem).wait()
        pltpu.sync_copy(rown_v, hn_hbm.at[pl.ds(b, CH_N)])

    ebase = wid * E_PER_W

    @pl.loop(0, E_PER_W // CH_E)
    def _(k):
        b = ebase + k * CH_E
        pltpu.sync_copy(ea_hbm.at[pl.ds(b, CH_E)], idxe_v)
        pltpu.async_copy(etab_hbm.at[idxe_v], rowe_v, sem).wait()
        pltpu.sync_copy(rowe_v, he_hbm.at[pl.ds(b, CH_E)])


@functools.partial(
    pl.kernel,
    out_type=(
        jax.ShapeDtypeStruct((N_EDGES, D), jnp.float32),
        jax.ShapeDtypeStruct((NP, D), jnp.float32),
    ),
    mesh=_mesh,
    scratch_types=[
        pltpu.VMEM((CH_E,), jnp.int32),
        pltpu.VMEM((CH_E, D), jnp.float32),
        pltpu.VMEM((CH_N,), jnp.int32),
        pltpu.VMEM((CH_N, D), jnp.float32),
        pltpu.SemaphoreType.DMA,
    ],
    compiler_params=_sc_params,
)
def _gather_kernel(*refs):
    _gather_body(*refs)


def _node_bn_body(h_ref, g_ref, b_ref, out_ref):
    h = h_ref[...]
    mask = (
        lax.broadcasted_iota(jnp.int32, (NP, 1), 0) < N_NODES
    ).astype(jnp.float32)
    hm = h * mask
    s1 = jnp.sum(hm, axis=0, keepdims=True)
    s2 = jnp.sum(hm * hm, axis=0, keepdims=True)
    inv_n = jnp.float32(1.0 / N_NODES)
    mean = s1 * inv_n
    var = s2 * inv_n - mean * mean
    a = g_ref[...] * lax.rsqrt(var + EPS)
    b = b_ref[...] - mean * a
    out_ref[...] = h[:N_NODES, :] * a + b


def _node_bn(h_raw, gamma, beta):
    return pl.pallas_call(
        _node_bn_body,
        out_shape=jax.ShapeDtypeStruct((N_NODES, D), jnp.float32),
    )(h_raw, gamma.reshape(1, D), beta.reshape(1, D))


def kernel(x, edge_attr, node_table, edge_table,
           node_gamma, node_beta, edge_gamma, edge_beta):
    x = x.astype(jnp.int32)
    edge_attr = edge_attr.astype(jnp.int32)
    x_pad = jnp.pad(x, (0, NP - N_NODES))

    counts = _hist_kernel(edge_attr)
    norm_etab = _edge_table(counts, edge_table, edge_gamma, edge_beta)
    h_edge, h_node_raw = _gather_kernel(norm_etab, edge_attr, node_table, x_pad)
    h_node = _node_bn(h_node_raw, node_gamma, node_beta)
    rm = jlayout.Layout(major_to_minor=(0, 1))
    h_edge = jlayout.with_layout_constraint(h_edge, rm)
    h_node = jlayout.with_layout_constraint(h_node, rm)
    return (h_node, h_edge)
